# Initial kernel scaffold; baseline (speedup 1.0000x reference)
import jax
import jax.numpy as jnp
from jax.experimental import pallas as pl


def kernel(h, spans, W):
    raise NotImplementedError("write your pallas kernel here")


# speedup vs baseline: 12.1517x; 12.1517x over previous
import functools

import jax
import jax.numpy as jnp
from jax import lax
from jax.experimental import pallas as pl
from jax.experimental.pallas import tpu as pltpu
from jax.experimental.pallas import tpu_sc as plsc

B, S, D = 8, 2048, 768
NS, L = 256, 16
NC, NSC = 2, 16
NW = NC * NSC
SPW = (B * NS) // NW
G = 4
NG = SPW // G
CH = D // L


_BB = 2


def _logits_body(h_ref, w_ref, out_ref):
    for i in range(_BB):
        out_ref[i] = lax.dot_general(
            h_ref[i], w_ref[...],
            dimension_numbers=(((1,), (1,)), ((), ())),
            preferred_element_type=jnp.float32)


def _compute_logits(h, w):
    return pl.pallas_call(
        _logits_body,
        grid=(B // _BB,),
        in_specs=[
            pl.BlockSpec((_BB, S, D), lambda i: (i, 0, 0)),
            pl.BlockSpec((1, D), lambda i: (0, 0)),
        ],
        out_specs=pl.BlockSpec((_BB, S, 1), lambda i: (i, 0, 0)),
        out_shape=jax.ShapeDtypeStruct((B, S, 1), jnp.float32),
    )(h, w)


def _make_sc_pool():
    mesh = plsc.VectorSubcoreMesh(core_axis_name="c", subcore_axis_name="s")

    @functools.partial(
        pl.kernel,
        out_type=jax.ShapeDtypeStruct((B * NS, D), jnp.float32),
        mesh=mesh,
        compiler_params=pltpu.CompilerParams(needs_layout_passes=False),
        scratch_types=[
            pltpu.VMEM((SPW * L,), jnp.int32),
            pltpu.VMEM((S,), jnp.float32),
            pltpu.VMEM((G * L, D), jnp.float32),
            pltpu.VMEM((G * L, D), jnp.float32),
            pltpu.VMEM((G, D), jnp.float32),
            pltpu.VMEM((G, D), jnp.float32),
            pltpu.SemaphoreType.DMA,
            pltpu.SemaphoreType.DMA,
            pltpu.SemaphoreType.DMA,
            pltpu.SemaphoreType.DMA,
        ],
    )
    def sc_pool(h_hbm, idx_hbm, logit_hbm, out_hbm,
                idx_v, logit_v, rows0, rows1, out_v0, out_v1,
                sem0, sem1, semo0, semo1):
        wid = lax.axis_index("s") * NC + lax.axis_index("c")
        span_base = wid * SPW
        b = span_base // NS

        pltpu.sync_copy(idx_hbm.at[pl.ds(span_base * L, SPW * L)], idx_v)

        @plsc.parallel_loop(0, SPW, 1, unroll=4)
        def _globalize(k):
            sl = pl.ds(k * L, L)
            idx_v[sl] = idx_v[sl] + b * S

        def gather_src(g):
            return h_hbm.at[idx_v.at[pl.ds(g * (G * L), G * L)]]

        def gather_start(g, buf, sem):
            pltpu.async_copy(gather_src(g), buf, sem)

        def gather_wait(g, buf, sem):
            pltpu.make_async_copy(gather_src(g), buf, sem).wait()

        def out_dst(g):
            return out_hbm.at[pl.ds(span_base + g * G, G)]

        def out_start(g, buf, sem):
            pltpu.async_copy(buf, out_dst(g), sem)

        def out_wait(g, buf, sem):
            pltpu.make_async_copy(buf, out_dst(g), sem).wait()

        def compute_group(g, rows, out_v):
            @plsc.parallel_loop(0, G, 1)
            def span_body(j):
                idxv = idx_v[pl.ds((g * G + j) * L, L)]
                local = jnp.bitwise_and(idxv, S - 1)
                lg = plsc.load_gather(logit_v, [local])
                m = jnp.max(lg)
                e = jnp.exp(lg - m)
                w = e / jnp.sum(e)
                wbs = [
                    w.at[jnp.full((L,), l, jnp.int32)]
                    .get(mode="promise_in_bounds")
                    for l in range(L)
                ]
                r0 = j * L

                @plsc.parallel_loop(0, CH, 1, unroll=3)
                def chunk_body(c):
                    col = pl.ds(c * L, L)
                    a = [wbs[q] * rows[r0 + q, col] for q in range(4)]
                    for l in range(4, L, 4):
                        for q in range(4):
                            a[q] = a[q] + wbs[l + q] * rows[r0 + l + q, col]
                    out_v[j, col] = (a[0] + a[1]) + (a[2] + a[3])

        gather_start(0, rows0, sem0)
        pltpu.sync_copy(logit_hbm.at[pl.ds(b * S, S)], logit_v)

        def pair_body(gp, _):
            g0 = gp * 2
            gather_start(g0 + 1, rows1, sem1)
            gather_wait(g0, rows0, sem0)

            @pl.when(gp > 0)
            def _():
                out_wait(g0 - 2, out_v0, semo0)

            compute_group(g0, rows0, out_v0)
            out_start(g0, out_v0, semo0)

            @pl.when(g0 + 2 < NG)
            def _():
                gather_start(g0 + 2, rows0, sem0)

            gather_wait(g0 + 1, rows1, sem1)

            @pl.when(gp > 0)
            def _():
                out_wait(g0 - 1, out_v1, semo1)

            compute_group(g0 + 1, rows1, out_v1)
            out_start(g0 + 1, out_v1, semo1)
            return 0

        lax.fori_loop(0, NG // 2, pair_body, 0)
        out_wait(NG - 2, out_v0, semo0)
        out_wait(NG - 1, out_v1, semo1)

    return sc_pool


_sc_pool = _make_sc_pool()


def kernel(h, spans, W):
    logits = _compute_logits(h, W)
    pooled = _sc_pool(
        h.reshape(B * S, D),
        spans.reshape(B * NS * L),
        logits.reshape(B * S),
    )
    return pooled.reshape(B, NS, D)
